# baseline (speedup 1.0000x reference)
import functools

import jax
import jax.numpy as jnp
from jax import lax
from jax.experimental import pallas as pl
from jax.experimental.pallas import tpu as pltpu
from jax.experimental.pallas import tpu_sc as plsc

B, L = 4096, 20
EMB = 64
PAIR = 2 * EMB
C = 235
N = B * L

NUM_CORES = 2
NUM_SUBCORES = 16
NW = NUM_CORES * NUM_SUBCORES
CHUNK = 128
PER_W = N // NW
NCH = PER_W // CHUNK

BBLK = 512
NB = B // BBLK


def _gather_body(table_hbm, idx_hbm, out_hbm, idx_v, rows, sems):
    wid = lax.axis_index("s") * NUM_CORES + lax.axis_index("c")
    out_row0 = wid * PER_W
    pltpu.sync_copy(idx_hbm.at[wid], idx_v)
    copies = [None, None]
    copies[0] = pltpu.async_copy(table_hbm.at[idx_v.at[0]], rows[0], sems[0])
    for j in range(NCH):
        cur = j % 2
        nxt = (j + 1) % 2
        if j + 1 < NCH:
            copies[nxt] = pltpu.async_copy(
                table_hbm.at[idx_v.at[j + 1]], rows[nxt], sems[nxt])
        copies[cur].wait()
        pltpu.sync_copy(rows[cur], out_hbm.at[pl.ds(out_row0 + j * CHUNK, CHUNK)])


@functools.cache
def _make_sc_gather():
    return pl.kernel(
        _gather_body,
        out_type=jax.ShapeDtypeStruct((N, PAIR), jnp.float32),
        mesh=plsc.VectorSubcoreMesh(core_axis_name="c", subcore_axis_name="s"),
        scratch_types=[
            pltpu.VMEM((NCH, CHUNK), jnp.int32),
            [pltpu.VMEM((CHUNK, PAIR), jnp.float32),
             pltpu.VMEM((CHUNK, PAIR), jnp.float32)],
            [pltpu.SemaphoreType.DMA, pltpu.SemaphoreType.DMA],
        ],
    )


def _lane_mask(par_col):
    lane = lax.broadcasted_iota(jnp.int32, (1, PAIR), 1)
    return jnp.where(lane < EMB, 1.0 - par_col, par_col)


def _lse_kernel(emb_ref, par_ref, wt2_ref, out_ref, m_ref, s_ref):
    l = pl.program_id(0)
    i = pl.program_id(1)

    @pl.when(i == 0)
    def _init():
        m_ref[...] = jnp.full((1, C), -jnp.inf, dtype=jnp.float32)
        s_ref[...] = jnp.zeros((1, C), dtype=jnp.float32)

    par = par_ref[...].astype(jnp.float32)
    e2 = emb_ref[...] * _lane_mask(par)
    x = lax.dot_general(e2, wt2_ref[...], (((1,), (0,)), ((), ())),
                        preferred_element_type=jnp.float32)
    bm = jnp.max(x, axis=0, keepdims=True)
    bs = jnp.sum(jnp.exp(x - bm), axis=0, keepdims=True)
    m_old = m_ref[...]
    s_old = s_ref[...]
    m_new = jnp.maximum(m_old, bm)
    s_ref[...] = s_old * jnp.exp(m_old - m_new) + bs * jnp.exp(bm - m_new)
    m_ref[...] = m_new

    @pl.when(i == NB - 1)
    def _fin():
        out_ref[pl.ds(l, 1), :] = m_ref[...] + jnp.log(s_ref[...])


def _out_kernel(emb_hbm, par_hbm, wt2_ref, lse_ref, out_ref,
                ebuf, pbuf, esem, psem):
    i = pl.program_id(0)
    for l in range(L):
        r0 = l * B + i * BBLK
        pltpu.make_async_copy(
            emb_hbm.at[pl.ds(r0, BBLK)], ebuf.at[l], esem.at[l]).start()
        pltpu.make_async_copy(
            par_hbm.at[pl.ds(r0, BBLK)], pbuf.at[l], psem.at[l]).start()
    wt2 = wt2_ref[...]
    for l in range(L):
        r0 = l * B + i * BBLK
        pltpu.make_async_copy(
            emb_hbm.at[pl.ds(r0, BBLK)], ebuf.at[l], esem.at[l]).wait()
        pltpu.make_async_copy(
            par_hbm.at[pl.ds(r0, BBLK)], pbuf.at[l], psem.at[l]).wait()
        par = pbuf[l].astype(jnp.float32)
        e2 = ebuf[l] * _lane_mask(par)
        x = lax.dot_general(e2, wt2, (((1,), (0,)), ((), ())),
                            preferred_element_type=jnp.float32)
        lse_row = lse_ref[pl.ds(l, 1), :]
        out_ref[:, pl.ds(l, 1), :] = (x - lse_row)[:, None, :]


def kernel(input, emb_weight, fc_weight, fc_bias):
    idx_l = input.astype(jnp.int32).T.reshape(-1)
    idx_pair = lax.shift_right_logical(idx_l, 1).reshape(NW, NCH, CHUNK)
    parity = lax.bitwise_and(idx_l, 1).astype(jnp.int8).reshape(N, 1)
    table2 = emb_weight.reshape(-1, PAIR)

    pairs = _make_sc_gather()(table2, idx_pair)
    wt2 = jnp.concatenate([fc_weight.T, fc_weight.T], axis=0)

    lse = pl.pallas_call(
        _lse_kernel,
        grid=(L, NB),
        in_specs=[
            pl.BlockSpec((BBLK, PAIR), lambda l, i: (l * NB + i, 0)),
            pl.BlockSpec((BBLK, 1), lambda l, i: (l * NB + i, 0)),
            pl.BlockSpec((PAIR, C), lambda l, i: (0, 0)),
        ],
        out_specs=pl.BlockSpec((L, C), lambda l, i: (0, 0)),
        out_shape=jax.ShapeDtypeStruct((L, C), jnp.float32),
        scratch_shapes=[
            pltpu.VMEM((1, C), jnp.float32),
            pltpu.VMEM((1, C), jnp.float32),
        ],
    )(pairs, parity, wt2)

    out = pl.pallas_call(
        _out_kernel,
        grid=(NB,),
        in_specs=[
            pl.BlockSpec(memory_space=pl.ANY),
            pl.BlockSpec(memory_space=pl.ANY),
            pl.BlockSpec((PAIR, C), lambda i: (0, 0)),
            pl.BlockSpec((L, C), lambda i: (0, 0)),
        ],
        out_specs=pl.BlockSpec((BBLK, L, C), lambda i: (i, 0, 0)),
        out_shape=jax.ShapeDtypeStruct((B, L, C), jnp.float32),
        scratch_shapes=[
            pltpu.VMEM((L, BBLK, PAIR), jnp.float32),
            pltpu.VMEM((L, BBLK, 1), jnp.int8),
            pltpu.SemaphoreType.DMA((L,)),
            pltpu.SemaphoreType.DMA((L,)),
        ],
    )(pairs, parity, wt2, lse)
    return out


# speedup vs baseline: 1.4671x; 1.4671x over previous
import functools

import jax
import jax.numpy as jnp
from jax import lax
from jax.experimental import pallas as pl
from jax.experimental.pallas import tpu as pltpu
from jax.experimental.pallas import tpu_sc as plsc

B, L = 4096, 20
EMB = 64
C = 235
N = B * L
GRP = 8

NUM_CORES = 2
NUM_SUBCORES = 16
NW = NUM_CORES * NUM_SUBCORES
PER_W = N // NW
IDXCOLS = 128
CHUNK = 32
NCH = PER_W // CHUNK

BBLK = 512
NB = B // BBLK


def _gather_body(table_hbm, idx_hbm, out_hbm,
                 idxbuf, groups, comp, gsem, ssem):
    wid = lax.axis_index("s") * NUM_CORES + lax.axis_index("c")
    row0 = wid * PER_W
    pltpu.sync_copy(idx_hbm.at[wid], idxbuf)

    def chunk_vecs(c):
        row = lax.div(c, IDXCOLS // CHUNK)
        off = lax.rem(c, IDXCOLS // CHUNK) * CHUNK
        return (idxbuf[row, pl.ds(off, 16)], idxbuf[row, pl.ds(off + 16, 16)])

    def fire(c, u):
        va, vb = chunk_vecs(c)
        ga = lax.shift_right_logical(va, 3)
        gb = lax.shift_right_logical(vb, 3)
        for k in range(16):
            pltpu.async_copy(table_hbm.at[ga[k]], groups[u].at[k], gsem[u])
            pltpu.async_copy(table_hbm.at[gb[k]], groups[u].at[16 + k], gsem[u])

    def drain(c, u):
        for k in range(CHUNK):
            pltpu.make_async_copy(
                table_hbm.at[0], groups[u].at[k], gsem[u]).wait()
        va, vb = chunk_vecs(c)
        ra = lax.bitwise_and(va, 7)
        rb = lax.bitwise_and(vb, 7)
        for k in range(16):
            for q in range(EMB // 16):
                comp[u][k, pl.ds(q * 16, 16)] = (
                    groups[u][k, ra[k], pl.ds(q * 16, 16)])
                comp[u][16 + k, pl.ds(q * 16, 16)] = (
                    groups[u][16 + k, rb[k], pl.ds(q * 16, 16)])
        pltpu.sync_copy(comp[u], out_hbm.at[pl.ds(row0 + c * CHUNK, CHUNK)])

    fire(0, 0)

    def body(h, carry):
        c0 = 2 * h
        fire(c0 + 1, 1)
        drain(c0, 0)

        @pl.when(c0 + 2 < NCH)
        def _next():
            fire(c0 + 2, 0)

        drain(c0 + 1, 1)
        return carry

    lax.fori_loop(0, NCH // 2, body, 0)


@functools.cache
def _make_sc_gather():
    return pl.kernel(
        _gather_body,
        out_type=jax.ShapeDtypeStruct((N, EMB), jnp.float32),
        mesh=plsc.VectorSubcoreMesh(core_axis_name="c", subcore_axis_name="s"),
        scratch_types=[
            pltpu.VMEM((PER_W // IDXCOLS, IDXCOLS), jnp.int32),
            [pltpu.VMEM((CHUNK, GRP, EMB), jnp.float32)] * 2,
            [pltpu.VMEM((CHUNK, EMB), jnp.float32)] * 2,
            [pltpu.SemaphoreType.DMA] * 2,
            [pltpu.SemaphoreType.DMA] * 2,
        ],
    )


def _lse_kernel(emb_ref, wt_ref, out_ref, m_ref, s_ref):
    l = pl.program_id(0)
    i = pl.program_id(1)

    @pl.when(i == 0)
    def _init():
        m_ref[...] = jnp.full((1, C), -jnp.inf, dtype=jnp.float32)
        s_ref[...] = jnp.zeros((1, C), dtype=jnp.float32)

    x = lax.dot_general(emb_ref[...], wt_ref[...], (((1,), (0,)), ((), ())),
                        preferred_element_type=jnp.float32)
    bm = jnp.max(x, axis=0, keepdims=True)
    bs = jnp.sum(jnp.exp(x - bm), axis=0, keepdims=True)
    m_old = m_ref[...]
    s_old = s_ref[...]
    m_new = jnp.maximum(m_old, bm)
    s_ref[...] = s_old * jnp.exp(m_old - m_new) + bs * jnp.exp(bm - m_new)
    m_ref[...] = m_new

    @pl.when(i == NB - 1)
    def _fin():
        out_ref[pl.ds(l, 1), :] = m_ref[...] + jnp.log(s_ref[...])


def _out_kernel(emb_hbm, wt_ref, lse_ref, out_ref, ebuf, esem):
    i = pl.program_id(0)
    for l in range(L):
        pltpu.make_async_copy(
            emb_hbm.at[pl.ds(l * B + i * BBLK, BBLK)], ebuf.at[l],
            esem.at[l]).start()
    wt = wt_ref[...]
    for l in range(L):
        pltpu.make_async_copy(
            emb_hbm.at[pl.ds(l * B + i * BBLK, BBLK)], ebuf.at[l],
            esem.at[l]).wait()
        x = lax.dot_general(ebuf[l], wt, (((1,), (0,)), ((), ())),
                            preferred_element_type=jnp.float32)
        lse_row = lse_ref[pl.ds(l, 1), :]
        out_ref[:, pl.ds(l, 1), :] = (x - lse_row)[:, None, :]


def kernel(input, emb_weight, fc_weight, fc_bias):
    idx_l = input.astype(jnp.int32).T.reshape(-1)
    idx3 = idx_l.reshape(NW, PER_W // IDXCOLS, IDXCOLS)
    table3 = emb_weight.reshape(-1, GRP, EMB)

    emb_l = _make_sc_gather()(table3, idx3)
    wt = fc_weight.T

    lse = pl.pallas_call(
        _lse_kernel,
        grid=(L, NB),
        in_specs=[
            pl.BlockSpec((BBLK, EMB), lambda l, i: (l * NB + i, 0)),
            pl.BlockSpec((EMB, C), lambda l, i: (0, 0)),
        ],
        out_specs=pl.BlockSpec((L, C), lambda l, i: (0, 0)),
        out_shape=jax.ShapeDtypeStruct((L, C), jnp.float32),
        scratch_shapes=[
            pltpu.VMEM((1, C), jnp.float32),
            pltpu.VMEM((1, C), jnp.float32),
        ],
    )(emb_l, wt)

    out = pl.pallas_call(
        _out_kernel,
        grid=(NB,),
        in_specs=[
            pl.BlockSpec(memory_space=pl.ANY),
            pl.BlockSpec((EMB, C), lambda i: (0, 0)),
            pl.BlockSpec((L, C), lambda i: (0, 0)),
        ],
        out_specs=pl.BlockSpec((BBLK, L, C), lambda i: (i, 0, 0)),
        out_shape=jax.ShapeDtypeStruct((B, L, C), jnp.float32),
        scratch_shapes=[
            pltpu.VMEM((L, BBLK, EMB), jnp.float32),
            pltpu.SemaphoreType.DMA((L,)),
        ],
    )(emb_l, wt, lse)
    return out
